# baseline (speedup 1.0000x reference)
import functools

import numpy as np
import jax
import jax.numpy as jnp
from jax import lax
from jax.experimental import pallas as pl
from jax.experimental.pallas import tpu as pltpu
from jax.experimental.pallas import tpu_sc as plsc

B, L = 4096, 50
N = B * L
V = 100000
D_P = 96
D_MODEL = 384

NW = 32
PER_W = N // NW
CH = 128
NCH = PER_W // CH

MB = 2000
TB = 1024


def _table_matmul_body(e_r, w_r, o0_r, o1_r, o2_r):
    x = jnp.dot(e_r[...], w_r[...], preferred_element_type=jnp.float32)
    o0_r[...] = x[:, 0:128]
    o1_r[...] = x[:, 128:256]
    o2_r[...] = x[:, 256:384]


def _tc_table_matmul(E, W):
    return pl.pallas_call(
        _table_matmul_body,
        grid=(V // MB,),
        in_specs=[
            pl.BlockSpec((MB, D_P), lambda i: (i, 0)),
            pl.BlockSpec((D_P, D_MODEL), lambda i: (0, 0)),
        ],
        out_specs=[pl.BlockSpec((MB, 128), lambda i: (i, 0))] * 3,
        out_shape=[jax.ShapeDtypeStruct((V, 128), jnp.float32)] * 3,
    )(E, W)


def _sc_gather_add(Ap3, Ab3, idx_p, idx_b):
    mesh = plsc.VectorSubcoreMesh(core_axis_name="c", subcore_axis_name="s")

    @functools.partial(
        pl.kernel,
        mesh=mesh,
        out_type=tuple(jax.ShapeDtypeStruct((N, 128), jnp.float32)
                       for _ in range(3)),
        scratch_types=[
            pltpu.VMEM((NCH, CH), jnp.int32),
            pltpu.VMEM((NCH, CH), jnp.int32),
            pltpu.VMEM((CH, 128), jnp.float32),
            pltpu.VMEM((CH, 128), jnp.float32),
            pltpu.VMEM((CH, 128), jnp.float32),
            pltpu.SemaphoreType.DMA,
        ],
    )
    def k(ap0, ap1, ap2, ab0, ab1, ab2, ip_hbm, ib_hbm, o0, o1, o2,
          ipv, ibv, b0, b1, b2, sem):
        cid = lax.axis_index("c")
        sid = lax.axis_index("s")
        wid = sid * 2 + cid
        base = wid * PER_W
        pltpu.sync_copy(ip_hbm.at[wid], ipv)
        pltpu.sync_copy(ib_hbm.at[wid], ibv)
        aps = (ap0, ap1, ap2)
        abs_ = (ab0, ab1, ab2)
        outs = (o0, o1, o2)
        bufs = (b0, b1, b2)

        def body(j, carry):
            idx = ipv.at[j]
            cps = [pltpu.async_copy(aps[c].at[idx], bufs[c], sem)
                   for c in range(3)]
            for cp in cps:
                cp.wait()
            idxb = ibv.at[j]
            cps = [pltpu.async_copy(abs_[c].at[idxb], bufs[c], sem, add=True)
                   for c in range(3)]
            for cp in cps:
                cp.wait()
            row0 = base + j * CH
            for c in range(3):
                pltpu.sync_copy(bufs[c], outs[c].at[pl.ds(row0, CH)])
            return carry

        lax.fori_loop(0, NCH, body, 0)

    return k(*Ap3, *Ab3, idx_p, idx_b)


_INV_SQRT2 = np.float32(1.0 / np.sqrt(2.0))


def _mlp_body(za_r, zb_r, zc_r, cp_r, sid_r, w1c_r, tsm_r, w2_r, b2_r, out_r):
    z1 = jnp.concatenate([za_r[...], zb_r[...], zc_r[...]], axis=-1)
    x = z1 + jnp.dot(cp_r[...], w1c_r[...],
                     preferred_element_type=jnp.float32)
    oh = (lax.broadcasted_iota(jnp.int32, (TB, 64), 1) == sid_r[...]
          ).astype(jnp.float32)
    x = x + jnp.dot(oh, tsm_r[...], preferred_element_type=jnp.float32)
    x = 0.5 * x * (1.0 + lax.erf(x * _INV_SQRT2))
    out_r[...] = (jnp.dot(x, w2_r[...], preferred_element_type=jnp.float32)
                  + b2_r[...])


def _tc_mlp(z1a, z1b, z1c, cp, sidx, W1c, Tsm, W2, b2):
    return pl.pallas_call(
        _mlp_body,
        grid=(N // TB,),
        in_specs=[
            pl.BlockSpec((TB, 128), lambda i: (i, 0)),
            pl.BlockSpec((TB, 128), lambda i: (i, 0)),
            pl.BlockSpec((TB, 128), lambda i: (i, 0)),
            pl.BlockSpec((TB, 21), lambda i: (i, 0)),
            pl.BlockSpec((TB, 1), lambda i: (i, 0)),
            pl.BlockSpec((21, D_MODEL), lambda i: (0, 0)),
            pl.BlockSpec((64, D_MODEL), lambda i: (0, 0)),
            pl.BlockSpec((D_MODEL, D_MODEL), lambda i: (0, 0)),
            pl.BlockSpec((1, D_MODEL), lambda i: (0, 0)),
        ],
        out_specs=pl.BlockSpec((TB, D_MODEL), lambda i: (i, 0)),
        out_shape=jax.ShapeDtypeStruct((N, D_MODEL), jnp.float32),
    )(z1a, z1b, z1c, cp, sidx, W1c, Tsm, W2, b2)


def kernel(pitcher_id, batter_id, p_throws_id, stand_id, inning_topbot_id,
           cont, profile, E_pitcher, E_batter, E_pthrows, E_stand, E_topbot,
           W1, b1, W2, b2):
    pid = pitcher_id.astype(jnp.int32).T.reshape(NW, NCH, CH)
    bid = batter_id.astype(jnp.int32).T.reshape(NW, NCH, CH)

    Ap3 = _tc_table_matmul(E_pitcher, W1[0:96])
    Ab3 = _tc_table_matmul(E_batter, W1[96:192])
    z1a, z1b, z1c = _sc_gather_add(Ap3, Ab3, pid, bid)

    sidx = (p_throws_id.astype(jnp.int32) * 16
            + stand_id.astype(jnp.int32) * 4
            + inning_topbot_id.astype(jnp.int32)).T.reshape(N, 1)
    cp = jnp.concatenate([cont, profile], axis=-1).transpose(1, 0, 2).reshape(N, 21)

    Tsm = (jnp.dot(E_pthrows, W1[192:196])[:, None, None, :]
           + jnp.dot(E_stand, W1[196:200])[None, :, None, :]
           + jnp.dot(E_topbot, W1[200:204])[None, None, :, :]
           + b1[None, None, None, :]).reshape(64, D_MODEL)

    out = _tc_mlp(z1a, z1b, z1c, cp, sidx, W1[204:225], Tsm, W2, b2[None, :])
    return out.reshape(L, B, D_MODEL).transpose(1, 0, 2)


# speedup vs baseline: 1.1172x; 1.1172x over previous
import functools

import numpy as np
import jax
import jax.numpy as jnp
from jax import lax
from jax.experimental import pallas as pl
from jax.experimental.pallas import tpu as pltpu
from jax.experimental.pallas import tpu_sc as plsc

B, L = 4096, 50
N = B * L
V = 100000
D_P = 96
D_MODEL = 384

NW = 32
CH = 128

S = 5
NSEG = N // S
PER_W = NSEG // NW
NCH = PER_W // CH

MB = 2000
TB = 1024
SEG_BLOCKS = NSEG // TB


def _table_matmul_body(e_r, w_r, o0_r, o1_r, o2_r):
    x = jnp.dot(e_r[...], w_r[...], preferred_element_type=jnp.float32)
    o0_r[...] = x[:, 0:128]
    o1_r[...] = x[:, 128:256]
    o2_r[...] = x[:, 256:384]


def _tc_table_matmul(E, W):
    return pl.pallas_call(
        _table_matmul_body,
        grid=(V // MB,),
        in_specs=[
            pl.BlockSpec((MB, D_P), lambda i: (i, 0)),
            pl.BlockSpec((D_P, D_MODEL), lambda i: (0, 0)),
        ],
        out_specs=[pl.BlockSpec((MB, 128), lambda i: (i, 0))] * 3,
        out_shape=[jax.ShapeDtypeStruct((V, 128), jnp.float32)] * 3,
    )(E, W)


def _sc_gather_add(Ap3, Ab3, idx_p, idx_b):
    mesh = plsc.VectorSubcoreMesh(core_axis_name="c", subcore_axis_name="s")

    @functools.partial(
        pl.kernel,
        mesh=mesh,
        out_type=tuple(jax.ShapeDtypeStruct((NSEG, 128), jnp.float32)
                       for _ in range(3)),
        scratch_types=[
            pltpu.VMEM((NCH, CH), jnp.int32),
            pltpu.VMEM((NCH, CH), jnp.int32),
            pltpu.VMEM((CH, 128), jnp.float32),
            pltpu.VMEM((CH, 128), jnp.float32),
            pltpu.VMEM((CH, 128), jnp.float32),
            pltpu.SemaphoreType.DMA,
        ],
    )
    def k(ap0, ap1, ap2, ab0, ab1, ab2, ip_hbm, ib_hbm, o0, o1, o2,
          ipv, ibv, b0, b1, b2, sem):
        cid = lax.axis_index("c")
        sid = lax.axis_index("s")
        wid = sid * 2 + cid
        base = wid * PER_W
        pltpu.sync_copy(ip_hbm.at[wid], ipv)
        pltpu.sync_copy(ib_hbm.at[wid], ibv)
        aps = (ap0, ap1, ap2)
        abs_ = (ab0, ab1, ab2)
        outs = (o0, o1, o2)
        bufs = (b0, b1, b2)

        def body(j, carry):
            idx = ipv.at[j]
            cps = [pltpu.async_copy(aps[c].at[idx], bufs[c], sem)
                   for c in range(3)]
            for cp in cps:
                cp.wait()
            idxb = ibv.at[j]
            cps = [pltpu.async_copy(abs_[c].at[idxb], bufs[c], sem, add=True)
                   for c in range(3)]
            for cp in cps:
                cp.wait()
            row0 = base + j * CH
            for c in range(3):
                pltpu.sync_copy(bufs[c], outs[c].at[pl.ds(row0, CH)])
            return carry

        lax.fori_loop(0, NCH, body, 0)

    return k(*Ap3, *Ab3, idx_p, idx_b)


_INV_SQRT2 = np.float32(1.0 / np.sqrt(2.0))


def _mlp_body0(za_r, zb_r, zc_r, cp_r, sid_r, w1c_r, tsm_r, w2_r, b2_r,
               out_r):
    z1 = jnp.concatenate([za_r[...], zb_r[...], zc_r[...]], axis=-1)
    x = z1 + jnp.dot(cp_r[...], w1c_r[...],
                     preferred_element_type=jnp.float32)
    oh = (lax.broadcasted_iota(jnp.int32, (TB, 64), 1) == sid_r[...]
          ).astype(jnp.float32)
    x = x + jnp.dot(oh, tsm_r[...], preferred_element_type=jnp.float32)
    x = 0.5 * x * (1.0 + lax.erf(x * _INV_SQRT2))
    out_r[...] = (jnp.dot(x, w2_r[...], preferred_element_type=jnp.float32)
                  + b2_r[...])


def _mlp_body(prev_r, *rest):
    del prev_r
    _mlp_body0(*rest)


def _tc_mlp_seg(s, prev, z1a, z1b, z1c, cp, sidx, W1c, Tsm, W2, b2):
    base = s * SEG_BLOCKS
    specs = [
        pl.BlockSpec((TB, 128), lambda i: (i, 0)),
        pl.BlockSpec((TB, 128), lambda i: (i, 0)),
        pl.BlockSpec((TB, 128), lambda i: (i, 0)),
        pl.BlockSpec((TB, 21), lambda i: (base + i, 0)),
        pl.BlockSpec((TB, 1), lambda i: (base + i, 0)),
        pl.BlockSpec((21, D_MODEL), lambda i: (0, 0)),
        pl.BlockSpec((64, D_MODEL), lambda i: (0, 0)),
        pl.BlockSpec((D_MODEL, D_MODEL), lambda i: (0, 0)),
        pl.BlockSpec((1, D_MODEL), lambda i: (0, 0)),
    ]
    args = (z1a, z1b, z1c, cp, sidx, W1c, Tsm, W2, b2)
    if prev is None:
        body, aliases = _mlp_body0, {}
    else:
        body, aliases = _mlp_body, {0: 0}
        specs = [pl.BlockSpec(memory_space=pl.ANY)] + specs
        args = (prev,) + args
    return pl.pallas_call(
        body,
        grid=(SEG_BLOCKS,),
        in_specs=specs,
        out_specs=pl.BlockSpec((TB, D_MODEL), lambda i: (base + i, 0)),
        out_shape=jax.ShapeDtypeStruct((N, D_MODEL), jnp.float32),
        input_output_aliases=aliases,
    )(*args)


def kernel(pitcher_id, batter_id, p_throws_id, stand_id, inning_topbot_id,
           cont, profile, E_pitcher, E_batter, E_pthrows, E_stand, E_topbot,
           W1, b1, W2, b2):
    pid = pitcher_id.astype(jnp.int32).T.reshape(S, NW, NCH, CH)
    bid = batter_id.astype(jnp.int32).T.reshape(S, NW, NCH, CH)

    Ap3 = _tc_table_matmul(E_pitcher, W1[0:96])
    Ab3 = _tc_table_matmul(E_batter, W1[96:192])

    sidx = (p_throws_id.astype(jnp.int32) * 16
            + stand_id.astype(jnp.int32) * 4
            + inning_topbot_id.astype(jnp.int32)).T.reshape(N, 1)
    cp = jnp.concatenate([cont, profile], axis=-1).transpose(1, 0, 2).reshape(N, 21)

    Tsm = (jnp.dot(E_pthrows, W1[192:196])[:, None, None, :]
           + jnp.dot(E_stand, W1[196:200])[None, :, None, :]
           + jnp.dot(E_topbot, W1[200:204])[None, None, :, :]
           + b1[None, None, None, :]).reshape(64, D_MODEL)

    W1c = W1[204:225]
    b2r = b2[None, :]
    z1 = [_sc_gather_add(Ap3, Ab3, pid[s], bid[s]) for s in range(S)]
    out = None
    for s in range(S):
        z1a, z1b, z1c = z1[s]
        out = _tc_mlp_seg(s, out, z1a, z1b, z1c, cp, sidx, W1c, Tsm, W2, b2r)
    return out.reshape(L, B, D_MODEL).transpose(1, 0, 2)
